# baseline (speedup 1.0000x reference)
import jax
import jax.numpy as jnp
import numpy as np
from jax import lax
from jax.experimental import pallas as pl
from jax.experimental.pallas import tpu as pltpu
from jax.experimental.pallas import tpu_sc as plsc

DIM = 1024
INTER = 512
E = 8
T = 2048
A = 2 * T
B = 128
NB = 40
NBPAD = 48
R = NB * B
NSC = 16
APT = A // NSC
NW = 32
RPW = R // NW
JPW = A // NW

_NEG = -1e30


def _gate_kernel(x_ref, gw_ref, gb_ref, pm_ref, eidx_ref, wgt_ref):
    xf = x_ref[...]
    scores = jax.lax.dot_general(
        xf.astype(jnp.bfloat16), gw_ref[...].astype(jnp.bfloat16),
        (((1,), (1,)), ((), ())), preferred_element_type=jnp.float32)
    s = jax.nn.sigmoid(scores)
    sb = s + gb_ref[...]
    lane = jax.lax.broadcasted_iota(jnp.int32, (T, E), 1)
    gid = lane // 2
    glane = sb + jax.lax.dot_general(
        sb, pm_ref[...], (((1,), (0,)), ((), ())),
        preferred_element_type=jnp.float32,
        precision=jax.lax.Precision.HIGHEST)
    m1 = jnp.max(glane, axis=1, keepdims=True)
    g1 = jnp.min(jnp.where(glane >= m1, gid, 99), axis=1, keepdims=True)
    gl2 = jnp.where(gid == g1, _NEG, glane)
    m2 = jnp.max(gl2, axis=1, keepdims=True)
    g2 = jnp.min(jnp.where(gl2 >= m2, gid, 99), axis=1, keepdims=True)
    keep = (gid == g1) | (gid == g2)
    sk = jnp.where(keep, sb, _NEG)
    v1 = jnp.max(sk, axis=1, keepdims=True)
    e1 = jnp.min(jnp.where(sk >= v1, lane, 99), axis=1, keepdims=True)
    sk2 = jnp.where(lane == e1, _NEG, sk)
    v2 = jnp.max(sk2, axis=1, keepdims=True)
    e2 = jnp.min(jnp.where(sk2 >= v2, lane, 99), axis=1, keepdims=True)
    w1 = jnp.sum(jnp.where(lane == e1, s, 0.0), axis=1, keepdims=True)
    w2 = jnp.sum(jnp.where(lane == e2, s, 0.0), axis=1, keepdims=True)
    norm = w1 + w2
    eidx_ref[...] = jnp.concatenate([e1, e2], axis=1)
    wgt_ref[...] = jnp.concatenate([w1 / norm, w2 / norm], axis=1)


_PAIR_M = np.zeros((E, E), np.float32)
for _e in range(E):
    _PAIR_M[_e ^ 1, _e] = 1.0


def _route_body(eidx_hbm, pos_hbm, stok_hbm, bexp_hbm,
                eloc, rankv, destv, tokv, histv, allh, basev, bexpv,
                shared_hist, sem):
    del sem
    s = lax.axis_index("s")
    iota = lax.iota(jnp.int32, 16)
    pltpu.sync_copy(eidx_hbm.at[pl.ds(s * APT, APT)], eloc)
    last = iota * 0 + 15
    gd = lax.GatherDimensionNumbers(offset_dims=(), collapsed_slice_dims=(0,),
                                    start_index_map=(0,))

    def _splat_last(v):
        return lax.gather(v, last[:, None], gd, (1,),
                          mode=lax.GatherScatterMode.PROMISE_IN_BOUNDS)
    run = [jnp.zeros((16,), jnp.int32) for _ in range(E)]
    for v in range(APT // 16):
        vec = eloc[pl.ds(v * 16, 16)]
        rank = jnp.zeros((16,), jnp.int32)
        for e in range(E):
            m = vec == e
            c = plsc.cumsum(jnp.where(m, 1, 0))
            rank = jnp.where(m, run[e] + c - 1, rank)
            run[e] = run[e] + _splat_last(c)
        rankv[pl.ds(v * 16, 16)] = rank
    hv = jnp.zeros((16,), jnp.int32)
    for e in range(E):
        hv = jnp.where(iota == e, run[e], hv)
    histv[...] = hv
    pltpu.sync_copy(histv, shared_hist.at[pl.ds(s * 16, 16)])
    plsc.subcore_barrier()
    pltpu.sync_copy(shared_hist, allh)
    tot = jnp.zeros((16,), jnp.int32)
    pre = jnp.zeros((16,), jnp.int32)
    for s2 in range(NSC):
        row = allh[pl.ds(s2 * 16, 16)]
        tot = tot + row
        pre = pre + jnp.where(s2 < s, row, 0)
    padcnt = (tot + (B - 1)) & (-B)
    sincl = plsc.cumsum(padcnt)
    sexcl = sincl - padcnt
    basev[...] = sexcl + pre
    for v in range(APT // 16):
        vec = eloc[pl.ds(v * 16, 16)]
        bvals = plsc.load_gather(basev, [vec])
        dest = bvals + rankv[pl.ds(v * 16, 16)]
        destv[v // 8, pl.ds((v % 8) * 16, 16)] = dest
        tokv[v // 8, pl.ds((v % 8) * 16, 16)] = (
            (s * APT + v * 16 + iota) >> 1)
    for j in range(APT // 128):
        pltpu.sync_copy(destv.at[j],
                        pos_hbm.at[pl.ds(s * APT + j * 128, 128)])
        pltpu.sync_copy(tokv.at[j], stok_hbm.at[destv.at[j]])
    @pl.when(s == 0)
    def _():
        blk_end = (sexcl + padcnt) >> 7
        for k in range(NBPAD // 16):
            bvec = iota + k * 16
            acc = jnp.zeros((16,), jnp.int32)
            for e in range(E):
                end_s = jnp.sum(jnp.where(iota == e, blk_end, 0))
                acc = acc + jnp.where(bvec >= end_s, 1, 0)
            tot_s = jnp.sum(jnp.where(iota == (E - 1), blk_end, 0))
            bexpv[pl.ds(k * 16, 16)] = jnp.where(bvec < tot_s, acc, -1)
        pltpu.sync_copy(bexpv, bexp_hbm)


def _route_sc(eidx_flat):
    mesh = plsc.VectorSubcoreMesh(core_axis_name="c", subcore_axis_name="s")
    f = pl.kernel(
        _route_body,
        out_type=(
            jax.ShapeDtypeStruct((A,), jnp.int32),
            jax.ShapeDtypeStruct((R,), jnp.int32),
            jax.ShapeDtypeStruct((NBPAD,), jnp.int32),
        ),
        mesh=mesh,
        scratch_types=[
            pltpu.VMEM((APT,), jnp.int32),
            pltpu.VMEM((APT,), jnp.int32),
            pltpu.VMEM((2, 128), jnp.int32),
            pltpu.VMEM((2, 128), jnp.int32),
            pltpu.VMEM((16,), jnp.int32),
            pltpu.VMEM((NSC * 16,), jnp.int32),
            pltpu.VMEM((16,), jnp.int32),
            pltpu.VMEM((NBPAD,), jnp.int32),
            pltpu.VMEM_SHARED((NSC * 16,), jnp.int32),
            pltpu.SemaphoreType.DMA,
        ],
        compiler_params=pltpu.CompilerParams(needs_layout_passes=False),
    )
    return f(eidx_flat)


def _gather_body(x_hbm, stok_hbm, xs_hbm, idxv, rows,
                 si0, si1, si2, so0, so1, so2):
    c = lax.axis_index("c")
    s = lax.axis_index("s")
    w = s * 2 + c
    base = w * RPW
    pltpu.sync_copy(stok_hbm.at[pl.ds(base, RPW)], idxv)
    for v in range(RPW // 16):
        vec = idxv[pl.ds(v * 16, 16)]
        idxv[pl.ds(v * 16, 16)] = jnp.minimum(jnp.maximum(vec, 0), T - 1)
    nch, ch = 4, RPW // 4
    sin = [si0, si1, si2]
    sout = [so0, so1, so2]
    ins = [None] * nch
    outs = [None] * nch
    for k in range(3):
        ins[k] = pltpu.async_copy(
            x_hbm.at[idxv.at[pl.ds(k * ch, ch)]],
            rows.at[pl.ds((k % 3) * ch, ch)], sin[k % 3])
    for k in range(nch):
        ins[k].wait()
        outs[k] = pltpu.async_copy(
            rows.at[pl.ds((k % 3) * ch, ch)],
            xs_hbm.at[pl.ds(base + k * ch, ch)], sout[k % 3])
        if k + 3 < nch:
            outs[k].wait()
            ins[k + 3] = pltpu.async_copy(
                x_hbm.at[idxv.at[pl.ds((k + 3) * ch, ch)]],
                rows.at[pl.ds(((k + 3) % 3) * ch, ch)], sin[(k + 3) % 3])
    for k in range(nch):
        if outs[k] is not None and k + 3 >= nch:
            outs[k].wait()


def _gather_sc(x2d, stok):
    mesh = plsc.VectorSubcoreMesh(core_axis_name="c", subcore_axis_name="s")
    f = pl.kernel(
        _gather_body,
        out_type=jax.ShapeDtypeStruct((R, DIM), jnp.float32),
        mesh=mesh,
        scratch_types=[
            pltpu.VMEM((RPW,), jnp.int32),
            pltpu.VMEM((3 * (RPW // 4), DIM), jnp.float32),
            pltpu.SemaphoreType.DMA,
            pltpu.SemaphoreType.DMA,
            pltpu.SemaphoreType.DMA,
            pltpu.SemaphoreType.DMA,
            pltpu.SemaphoreType.DMA,
            pltpu.SemaphoreType.DMA,
        ],
        compiler_params=pltpu.CompilerParams(needs_layout_passes=False),
    )
    return f(x2d, stok)


def _cgather_body(ys_hbm, pos_hbm, yg_hbm, idxv, rows,
                  si0, si1, si2, so0, so1, so2):
    c = lax.axis_index("c")
    s = lax.axis_index("s")
    w = s * 2 + c
    base = w * JPW
    pltpu.sync_copy(pos_hbm.at[pl.ds(base, JPW)], idxv)
    nch, ch = 4, JPW // 4
    sin = [si0, si1, si2]
    sout = [so0, so1, so2]
    ins = [None] * nch
    outs = [None] * nch
    for k in range(3):
        ins[k] = pltpu.async_copy(
            ys_hbm.at[idxv.at[pl.ds(k * ch, ch)]],
            rows.at[pl.ds((k % 3) * ch, ch)], sin[k % 3])
    for k in range(nch):
        ins[k].wait()
        outs[k] = pltpu.async_copy(
            rows.at[pl.ds((k % 3) * ch, ch)],
            yg_hbm.at[pl.ds(base + k * ch, ch)], sout[k % 3])
        if k + 3 < nch:
            outs[k].wait()
            ins[k + 3] = pltpu.async_copy(
                ys_hbm.at[idxv.at[pl.ds((k + 3) * ch, ch)]],
                rows.at[pl.ds(((k + 3) % 3) * ch, ch)], sin[(k + 3) % 3])
    for k in range(nch):
        if outs[k] is not None and k + 3 >= nch:
            outs[k].wait()


def _cgather_sc(ys, pos):
    mesh = plsc.VectorSubcoreMesh(core_axis_name="c", subcore_axis_name="s")
    f = pl.kernel(
        _cgather_body,
        out_type=jax.ShapeDtypeStruct((A, DIM), jnp.float32),
        mesh=mesh,
        scratch_types=[
            pltpu.VMEM((JPW,), jnp.int32),
            pltpu.VMEM((3 * (JPW // 4), DIM), jnp.float32),
            pltpu.SemaphoreType.DMA,
            pltpu.SemaphoreType.DMA,
            pltpu.SemaphoreType.DMA,
            pltpu.SemaphoreType.DMA,
            pltpu.SemaphoreType.DMA,
            pltpu.SemaphoreType.DMA,
        ],
        compiler_params=pltpu.CompilerParams(needs_layout_passes=False),
    )
    return f(ys, pos)


def _shared_kernel(x_ref, sw1_ref, sw3_ref, sw2_ref, z_ref):
    xf = x_ref[...].astype(jnp.bfloat16)
    h1 = jax.lax.dot_general(xf, sw1_ref[...].astype(jnp.bfloat16),
                             (((1,), (1,)), ((), ())),
                             preferred_element_type=jnp.float32)
    h3 = jax.lax.dot_general(xf, sw3_ref[...].astype(jnp.bfloat16),
                             (((1,), (1,)), ((), ())),
                             preferred_element_type=jnp.float32)
    h = (h1 * jax.nn.sigmoid(h1)) * h3
    z_ref[...] = jax.lax.dot_general(h.astype(jnp.bfloat16),
                                     sw2_ref[...].astype(jnp.bfloat16),
                                     (((1,), (1,)), ((), ())),
                                     preferred_element_type=jnp.float32)


def _group_kernel(bexp_ref, xs_ref, w1_ref, w3_ref, w2_ref, ys_ref):
    i = pl.program_id(0)
    be = bexp_ref[i]

    @pl.when(be >= 0)
    def _():
        xb = xs_ref[...].astype(jnp.bfloat16)
        h1 = jax.lax.dot_general(xb, w1_ref[0].astype(jnp.bfloat16),
                                 (((1,), (1,)), ((), ())),
                                 preferred_element_type=jnp.float32)
        h3 = jax.lax.dot_general(xb, w3_ref[0].astype(jnp.bfloat16),
                                 (((1,), (1,)), ((), ())),
                                 preferred_element_type=jnp.float32)
        h = (h1 * jax.nn.sigmoid(h1)) * h3
        ys_ref[...] = jax.lax.dot_general(
            h.astype(jnp.bfloat16), w2_ref[0].astype(jnp.bfloat16),
            (((1,), (1,)), ((), ())),
            preferred_element_type=jnp.float32)


def _combine_kernel(z_ref, yg_ref, wgt_ref, out_ref):
    yg = yg_ref[...]
    wa = wgt_ref[:, 0:1]
    wb = wgt_ref[:, 1:2]
    out_ref[...] = (z_ref[...] + wa * yg[:, :DIM] + wb * yg[:, DIM:])


def kernel(x, gate_w, gate_b, w1, b1, w2, b2, w3, b3,
           sw1, sb1, sw2, sb2, sw3, sb3):
    shape = x.shape
    xf = x.reshape(T, DIM)

    eidx, wgt = pl.pallas_call(
        _gate_kernel,
        out_shape=(jax.ShapeDtypeStruct((T, 2), jnp.int32),
                   jax.ShapeDtypeStruct((T, 2), jnp.float32)),
    )(xf, gate_w, gate_b.reshape(1, E), jnp.asarray(_PAIR_M))

    pos, stok, bexp = _route_sc(eidx.reshape(A))

    xs = _gather_sc(xf, stok)

    z = pl.pallas_call(
        _shared_kernel,
        out_shape=jax.ShapeDtypeStruct((T, DIM), jnp.float32),
    )(xf, sw1, sw3, sw2)

    ys = pl.pallas_call(
        _group_kernel,
        grid_spec=pltpu.PrefetchScalarGridSpec(
            num_scalar_prefetch=1,
            grid=(NB,),
            in_specs=[
                pl.BlockSpec((B, DIM), lambda i, bexp: (i, 0)),
                pl.BlockSpec((1, INTER, DIM),
                             lambda i, bexp: (jnp.maximum(bexp[i], 0), 0, 0)),
                pl.BlockSpec((1, INTER, DIM),
                             lambda i, bexp: (jnp.maximum(bexp[i], 0), 0, 0)),
                pl.BlockSpec((1, DIM, INTER),
                             lambda i, bexp: (jnp.maximum(bexp[i], 0), 0, 0)),
            ],
            out_specs=pl.BlockSpec((B, DIM), lambda i, bexp: (i, 0)),
        ),
        out_shape=jax.ShapeDtypeStruct((R, DIM), jnp.float32),
    )(bexp, xs, w1, w3, w2)

    yg = _cgather_sc(ys, pos)

    out = pl.pallas_call(
        _combine_kernel,
        grid=(8,),
        in_specs=[
            pl.BlockSpec((T // 8, DIM), lambda i: (i, 0)),
            pl.BlockSpec((T // 8, 2 * DIM), lambda i: (i, 0)),
            pl.BlockSpec((T // 8, 2), lambda i: (i, 0)),
        ],
        out_specs=pl.BlockSpec((T // 8, DIM), lambda i: (i, 0)),
        out_shape=jax.ShapeDtypeStruct((T, DIM), jnp.float32),
    )(z, yg.reshape(T, 2 * DIM), wgt)

    return out.reshape(shape)


# speedup vs baseline: 1.4198x; 1.4198x over previous
import jax
import jax.numpy as jnp
import numpy as np
from jax import lax
from jax.experimental import pallas as pl
from jax.experimental.pallas import tpu as pltpu
from jax.experimental.pallas import tpu_sc as plsc

DIM = 1024
INTER = 512
E = 8
T = 2048
A = 2 * T
B = 128
NB = 40
NBPAD = 48
R = NB * B
NSC = 16
APT = A // NSC
NW = 32
RPW = R // NW
JPW = A // NW

_NEG = -1e30


def _gate_kernel(x_ref, gw_ref, gb_ref, pm_ref, eidx_ref, wgt_ref):
    xf = x_ref[...]
    scores = jax.lax.dot_general(
        xf.astype(jnp.bfloat16), gw_ref[...].astype(jnp.bfloat16),
        (((1,), (1,)), ((), ())), preferred_element_type=jnp.float32)
    s = jax.nn.sigmoid(scores)
    sb = s + gb_ref[...]
    lane = jax.lax.broadcasted_iota(jnp.int32, (T, E), 1)
    gid = lane // 2
    glane = sb + jax.lax.dot_general(
        sb, pm_ref[...], (((1,), (0,)), ((), ())),
        preferred_element_type=jnp.float32,
        precision=jax.lax.Precision.HIGHEST)
    m1 = jnp.max(glane, axis=1, keepdims=True)
    g1 = jnp.min(jnp.where(glane >= m1, gid, 99), axis=1, keepdims=True)
    gl2 = jnp.where(gid == g1, _NEG, glane)
    m2 = jnp.max(gl2, axis=1, keepdims=True)
    g2 = jnp.min(jnp.where(gl2 >= m2, gid, 99), axis=1, keepdims=True)
    keep = (gid == g1) | (gid == g2)
    sk = jnp.where(keep, sb, _NEG)
    v1 = jnp.max(sk, axis=1, keepdims=True)
    e1 = jnp.min(jnp.where(sk >= v1, lane, 99), axis=1, keepdims=True)
    sk2 = jnp.where(lane == e1, _NEG, sk)
    v2 = jnp.max(sk2, axis=1, keepdims=True)
    e2 = jnp.min(jnp.where(sk2 >= v2, lane, 99), axis=1, keepdims=True)
    w1 = jnp.sum(jnp.where(lane == e1, s, 0.0), axis=1, keepdims=True)
    w2 = jnp.sum(jnp.where(lane == e2, s, 0.0), axis=1, keepdims=True)
    norm = w1 + w2
    eidx_ref[...] = jnp.concatenate([e1, e2], axis=1)
    wgt_ref[...] = jnp.concatenate([w1 / norm, w2 / norm], axis=1)


_PAIR_M = np.zeros((E, E), np.float32)
for _e in range(E):
    _PAIR_M[_e ^ 1, _e] = 1.0


def _route_body(eidx_hbm, x_hbm, pos_hbm, xs_hbm, bexp_hbm,
                eloc, destv, tokv, histv, allh, basev, bexpv, posv, rows,
                shared_hist, si0, si1, so0, so1):
    c = lax.axis_index("c")
    s = lax.axis_index("s")
    iota = lax.iota(jnp.int32, 16)
    last = iota * 0 + 15
    gd = lax.GatherDimensionNumbers(offset_dims=(), collapsed_slice_dims=(0,),
                                    start_index_map=(0,))

    def _splat_last(v):
        return lax.gather(v, last[:, None], gd, (1,),
                          mode=lax.GatherScatterMode.PROMISE_IN_BOUNDS)

    pltpu.sync_copy(eidx_hbm.at[pl.ds(s * APT, APT)], eloc)
    run = [jnp.zeros((16,), jnp.int32) for _ in range(E)]
    rankvecs = []
    for v in range(APT // 16):
        vec = eloc[pl.ds(v * 16, 16)]
        rank = jnp.zeros((16,), jnp.int32)
        for e in range(E):
            m = vec == e
            cs = plsc.cumsum(jnp.where(m, 1, 0))
            rank = jnp.where(m, run[e] + cs - 1, rank)
            run[e] = run[e] + _splat_last(cs)
        rankvecs.append(rank)
    hv = jnp.zeros((16,), jnp.int32)
    for e in range(E):
        hv = jnp.where(iota == e, run[e], hv)
    histv[...] = hv
    pltpu.sync_copy(histv, shared_hist.at[pl.ds(s * 16, 16)])
    plsc.subcore_barrier()
    pltpu.sync_copy(shared_hist, allh)
    tot = jnp.zeros((16,), jnp.int32)
    pre = jnp.zeros((16,), jnp.int32)
    for s2 in range(NSC):
        row = allh[pl.ds(s2 * 16, 16)]
        tot = tot + row
        pre = pre + jnp.where(s2 < s, row, 0)
    padcnt = (tot + (B - 1)) & (-B)
    sincl = plsc.cumsum(padcnt)
    sexcl = sincl - padcnt
    basev[...] = sexcl + pre
    for half in range(2):
        @pl.when(c == half)
        def _():
            for h in range(4):
                for q in range(2):
                    v = half * 8 + h * 2 + q
                    vec = eloc[pl.ds(v * 16, 16)]
                    bvals = plsc.load_gather(basev, [vec])
                    destv[h, pl.ds(q * 16, 16)] = bvals + rankvecs[v]
                    tokv[h, pl.ds(q * 16, 16)] = (
                        (s * APT + v * 16 + iota) >> 1)
    @pl.when(c == 0)
    def _():
        for v in range(APT // 16):
            vec = eloc[pl.ds(v * 16, 16)]
            bvals = plsc.load_gather(basev, [vec])
            posv[v // 2, pl.ds((v % 2) * 16, 16)] = bvals + rankvecs[v]
        pltpu.sync_copy(posv, pos_hbm.at[pl.ds(s * (APT // 32), APT // 32)])
    sin = [si0, si1]
    sout = [so0, so1]
    ins = [None] * 4
    outs = [None] * 4
    for h in range(2):
        ins[h] = pltpu.async_copy(x_hbm.at[tokv.at[h]],
                                  rows.at[pl.ds((h % 2) * 32, 32)], sin[h % 2])
    for h in range(4):
        ins[h].wait()
        outs[h] = pltpu.async_copy(rows.at[pl.ds((h % 2) * 32, 32)],
                                   xs_hbm.at[destv.at[h]], sout[h % 2])
        if h + 2 < 4:
            outs[h].wait()
            ins[h + 2] = pltpu.async_copy(
                x_hbm.at[tokv.at[h + 2]],
                rows.at[pl.ds(((h + 2) % 2) * 32, 32)], sin[(h + 2) % 2])
    outs[2].wait()
    outs[3].wait()
    @pl.when(jnp.logical_and(s == 0, c == 0))
    def _():
        blk_end = (sexcl + padcnt) >> 7
        for k in range(NBPAD // 16):
            bvec = iota + k * 16
            acc = jnp.zeros((16,), jnp.int32)
            for e in range(E):
                end_s = jnp.sum(jnp.where(iota == e, blk_end, 0))
                acc = acc + jnp.where(bvec >= end_s, 1, 0)
            tot_s = jnp.sum(jnp.where(iota == (E - 1), blk_end, 0))
            bexpv[pl.ds(k * 16, 16)] = jnp.where(bvec < tot_s, acc, -1)
        pltpu.sync_copy(bexpv, bexp_hbm)


def _route_sc(eidx_flat, x2d):
    mesh = plsc.VectorSubcoreMesh(core_axis_name="c", subcore_axis_name="s")
    f = pl.kernel(
        _route_body,
        out_type=(
            jax.ShapeDtypeStruct((A // 32, 32), jnp.int32),
            jax.ShapeDtypeStruct((R, DIM), jnp.float32),
            jax.ShapeDtypeStruct((NBPAD,), jnp.int32),
        ),
        mesh=mesh,
        scratch_types=[
            pltpu.VMEM((APT,), jnp.int32),
            pltpu.VMEM((4, 32), jnp.int32),
            pltpu.VMEM((4, 32), jnp.int32),
            pltpu.VMEM((16,), jnp.int32),
            pltpu.VMEM((NSC * 16,), jnp.int32),
            pltpu.VMEM((16,), jnp.int32),
            pltpu.VMEM((NBPAD,), jnp.int32),
            pltpu.VMEM((APT // 32, 32), jnp.int32),
            pltpu.VMEM((64, DIM), jnp.float32),
            pltpu.VMEM_SHARED((NSC * 16,), jnp.int32),
            pltpu.SemaphoreType.DMA,
            pltpu.SemaphoreType.DMA,
            pltpu.SemaphoreType.DMA,
            pltpu.SemaphoreType.DMA,
        ],
        compiler_params=pltpu.CompilerParams(needs_layout_passes=False),
    )
    return f(eidx_flat, x2d)


def _cgather_body(ys_hbm, pos_hbm, yg_hbm, idxv, rows,
                  si0, si1, si2, so0, so1, so2):
    c = lax.axis_index("c")
    s = lax.axis_index("s")
    w = s * 2 + c
    base = w * JPW
    pltpu.sync_copy(pos_hbm.at[pl.ds(base, JPW)], idxv)
    nch, ch = 4, JPW // 4
    sin = [si0, si1, si2]
    sout = [so0, so1, so2]
    ins = [None] * nch
    outs = [None] * nch
    for k in range(3):
        ins[k] = pltpu.async_copy(
            ys_hbm.at[idxv.at[pl.ds(k * ch, ch)]],
            rows.at[pl.ds((k % 3) * ch, ch)], sin[k % 3])
    for k in range(nch):
        ins[k].wait()
        outs[k] = pltpu.async_copy(
            rows.at[pl.ds((k % 3) * ch, ch)],
            yg_hbm.at[pl.ds(base + k * ch, ch)], sout[k % 3])
        if k + 3 < nch:
            outs[k].wait()
            ins[k + 3] = pltpu.async_copy(
                ys_hbm.at[idxv.at[pl.ds((k + 3) * ch, ch)]],
                rows.at[pl.ds(((k + 3) % 3) * ch, ch)], sin[(k + 3) % 3])
    for k in range(nch):
        if outs[k] is not None and k + 3 >= nch:
            outs[k].wait()


def _cgather_sc(ys, pos):
    mesh = plsc.VectorSubcoreMesh(core_axis_name="c", subcore_axis_name="s")
    f = pl.kernel(
        _cgather_body,
        out_type=jax.ShapeDtypeStruct((A, DIM), jnp.float32),
        mesh=mesh,
        scratch_types=[
            pltpu.VMEM((JPW,), jnp.int32),
            pltpu.VMEM((3 * (JPW // 4), DIM), jnp.float32),
            pltpu.SemaphoreType.DMA,
            pltpu.SemaphoreType.DMA,
            pltpu.SemaphoreType.DMA,
            pltpu.SemaphoreType.DMA,
            pltpu.SemaphoreType.DMA,
            pltpu.SemaphoreType.DMA,
        ],
        compiler_params=pltpu.CompilerParams(needs_layout_passes=False),
    )
    return f(ys, pos)


def _shared_kernel(x_ref, sw1_ref, sw3_ref, sw2_ref, z_ref):
    xf = x_ref[...].astype(jnp.bfloat16)
    h1 = jax.lax.dot_general(xf, sw1_ref[...].astype(jnp.bfloat16),
                             (((1,), (1,)), ((), ())),
                             preferred_element_type=jnp.float32)
    h3 = jax.lax.dot_general(xf, sw3_ref[...].astype(jnp.bfloat16),
                             (((1,), (1,)), ((), ())),
                             preferred_element_type=jnp.float32)
    h = (h1 * jax.nn.sigmoid(h1)) * h3
    z_ref[...] = jax.lax.dot_general(h.astype(jnp.bfloat16),
                                     sw2_ref[...].astype(jnp.bfloat16),
                                     (((1,), (1,)), ((), ())),
                                     preferred_element_type=jnp.float32)


def _group_kernel(bexp_ref, xs_ref, w1_ref, w3_ref, w2_ref, ys_ref):
    i = pl.program_id(0)
    be = bexp_ref[i]

    @pl.when(be >= 0)
    def _():
        xb = xs_ref[...].astype(jnp.bfloat16)
        h1 = jax.lax.dot_general(xb, w1_ref[0].astype(jnp.bfloat16),
                                 (((1,), (1,)), ((), ())),
                                 preferred_element_type=jnp.float32)
        h3 = jax.lax.dot_general(xb, w3_ref[0].astype(jnp.bfloat16),
                                 (((1,), (1,)), ((), ())),
                                 preferred_element_type=jnp.float32)
        h = (h1 * jax.nn.sigmoid(h1)) * h3
        ys_ref[...] = jax.lax.dot_general(
            h.astype(jnp.bfloat16), w2_ref[0].astype(jnp.bfloat16),
            (((1,), (1,)), ((), ())),
            preferred_element_type=jnp.float32)


def _combine_kernel(z_ref, yg_ref, wgt_ref, out_ref):
    yg = yg_ref[...]
    wa = wgt_ref[:, 0:1]
    wb = wgt_ref[:, 1:2]
    out_ref[...] = (z_ref[...] + wa * yg[:, :DIM] + wb * yg[:, DIM:])


def kernel(x, gate_w, gate_b, w1, b1, w2, b2, w3, b3,
           sw1, sb1, sw2, sb2, sw3, sb3):
    shape = x.shape
    xf = x.reshape(T, DIM)

    eidx, wgt = pl.pallas_call(
        _gate_kernel,
        out_shape=(jax.ShapeDtypeStruct((T, 2), jnp.int32),
                   jax.ShapeDtypeStruct((T, 2), jnp.float32)),
    )(xf, gate_w, gate_b.reshape(1, E), jnp.asarray(_PAIR_M))

    pos, xs, bexp = _route_sc(eidx.reshape(A), xf)
    pos = pos.reshape(A)

    z = pl.pallas_call(
        _shared_kernel,
        out_shape=jax.ShapeDtypeStruct((T, DIM), jnp.float32),
    )(xf, sw1, sw3, sw2)

    ys = pl.pallas_call(
        _group_kernel,
        grid_spec=pltpu.PrefetchScalarGridSpec(
            num_scalar_prefetch=1,
            grid=(NB,),
            in_specs=[
                pl.BlockSpec((B, DIM), lambda i, bexp: (i, 0)),
                pl.BlockSpec((1, INTER, DIM),
                             lambda i, bexp: (jnp.maximum(bexp[i], 0), 0, 0)),
                pl.BlockSpec((1, INTER, DIM),
                             lambda i, bexp: (jnp.maximum(bexp[i], 0), 0, 0)),
                pl.BlockSpec((1, DIM, INTER),
                             lambda i, bexp: (jnp.maximum(bexp[i], 0), 0, 0)),
            ],
            out_specs=pl.BlockSpec((B, DIM), lambda i, bexp: (i, 0)),
        ),
        out_shape=jax.ShapeDtypeStruct((R, DIM), jnp.float32),
    )(bexp, xs, w1, w3, w2)

    yg = _cgather_sc(ys, pos)

    out = pl.pallas_call(
        _combine_kernel,
        grid=(8,),
        in_specs=[
            pl.BlockSpec((T // 8, DIM), lambda i: (i, 0)),
            pl.BlockSpec((T // 8, 2 * DIM), lambda i: (i, 0)),
            pl.BlockSpec((T // 8, 2), lambda i: (i, 0)),
        ],
        out_specs=pl.BlockSpec((T // 8, DIM), lambda i: (i, 0)),
        out_shape=jax.ShapeDtypeStruct((T, DIM), jnp.float32),
    )(z, yg.reshape(T, 2 * DIM), wgt)

    return out.reshape(shape)
